# baseline (speedup 1.0000x reference)
import functools

import jax
import jax.numpy as jnp
from jax import lax
from jax.experimental import pallas as pl
from jax.experimental.pallas import tpu as pltpu
from jax.experimental.pallas import tpu_sc as plsc

NC = 2
NS = 16
NW = NC * NS
NBUF = 3
GB = 8


def _emb_body(bpw, s, table_hbm, idx_hbm, out_hbm, idx_v, rows_v, gsem, osem):
    cid = lax.axis_index("c")
    sid = lax.axis_index("s")
    wid = sid * NC + cid
    base_b = wid * bpw
    niter = bpw // GB

    pltpu.sync_copy(idx_hbm.at[pl.ds(base_b, bpw)], idx_v)

    def issue(u, slot):
        for g in range(GB):
            pltpu.async_copy(
                table_hbm.at[idx_v.at[u * GB + g]], rows_v.at[slot, g], gsem)

    for i in range(NBUF - 1):
        issue(i, i)

    def body(t, carry):
        slot = lax.rem(t, NBUF)
        for g in range(GB):
            pltpu.make_async_copy(
                table_hbm.at[idx_v.at[0]], rows_v.at[slot, g], gsem).wait()

        @pl.when(t >= 1)
        def _():
            pltpu.make_async_copy(
                rows_v.at[slot],
                out_hbm.at[pl.ds(0, GB), pl.ds(0, s), pl.ds(0, 64)],
                osem).wait()

        @pl.when(t + NBUF - 1 < niter)
        def _():
            issue(t + NBUF - 1, lax.rem(t + NBUF - 1, NBUF))

        pltpu.async_copy(
            rows_v.at[slot],
            out_hbm.at[pl.ds(base_b + t * GB, GB), pl.ds(0, s), pl.ds(0, 64)],
            osem)
        return carry

    lax.fori_loop(0, niter, body, 0)
    pltpu.make_async_copy(
        rows_v.at[0],
        out_hbm.at[pl.ds(0, GB), pl.ds(0, s), pl.ds(0, 64)],
        osem).wait()


def _lin_body(top_ref, bot_ref, out_ref):
    d = top_ref.shape[1]
    out_ref[:, :d] = top_ref[...]
    out_ref[:, d:] = bot_ref[...]


def _linearize_table(table):
    v, d = table.shape
    h = v // 2
    rb = 2000
    nblk = h // rb
    t_lin = pl.pallas_call(
        _lin_body,
        grid=(nblk,),
        in_specs=[
            pl.BlockSpec((rb, d), lambda i: (i, 0)),
            pl.BlockSpec((rb, d), lambda i: (i + nblk, 0)),
        ],
        out_specs=pl.BlockSpec((rb, 2 * d), lambda i: (i, 0)),
        out_shape=jax.ShapeDtypeStruct((h, 2 * d), jnp.float32),
    )(table, table)
    return t_lin.reshape(v, d)


def kernel(x, table):
    b, s = x.shape
    v, d = table.shape
    assert b % (NW * GB) == 0
    bpw = b // NW

    grid_kernel = pl.kernel(
        functools.partial(_emb_body, bpw, s),
        mesh=plsc.VectorSubcoreMesh(core_axis_name="c", subcore_axis_name="s"),
        out_type=jax.ShapeDtypeStruct((b, 56, 128), jnp.float32),
        scratch_types=[
            pltpu.VMEM((bpw, s), jnp.int32),
            pltpu.VMEM((NBUF, GB, s, d), jnp.float32),
            pltpu.SemaphoreType.DMA,
            pltpu.SemaphoreType.DMA,
        ],
        compiler_params=pltpu.CompilerParams(use_tc_tiling_on_sc=False),
    )

    xi = x.astype(jnp.int32)
    xr = jnp.where(xi < v // 2, 2 * xi, 2 * xi - (v - 1))
    out = grid_kernel(_linearize_table(table), xr)
    return out[:, :s, :d]


# speedup vs baseline: 1.0604x; 1.0604x over previous
import functools

import jax
import jax.numpy as jnp
from jax import lax
from jax.experimental import pallas as pl
from jax.experimental.pallas import tpu as pltpu
from jax.experimental.pallas import tpu_sc as plsc

NC = 2
NS = 16
NW = NC * NS
NBUF = 4
GB = 4


def _emb_body(bpw, s, table_hbm, idx_hbm, out_hbm, idx_v, rows_v, gsem, osem):
    cid = lax.axis_index("c")
    sid = lax.axis_index("s")
    wid = sid * NC + cid
    base_b = wid * bpw
    niter = bpw // GB

    pltpu.sync_copy(idx_hbm.at[pl.ds(base_b, bpw)], idx_v)

    def issue(u, slot):
        for g in range(GB):
            pltpu.async_copy(
                table_hbm.at[idx_v.at[u * GB + g]], rows_v.at[slot, g], gsem)

    for i in range(NBUF - 1):
        issue(i, i)

    def body(t, carry):
        slot = lax.rem(t, NBUF)
        for g in range(GB):
            pltpu.make_async_copy(
                table_hbm.at[idx_v.at[0]], rows_v.at[slot, g], gsem).wait()

        @pl.when(t >= 1)
        def _():
            pltpu.make_async_copy(
                rows_v.at[slot],
                out_hbm.at[pl.ds(0, GB), pl.ds(0, s), pl.ds(0, 64)],
                osem).wait()

        @pl.when(t + NBUF - 1 < niter)
        def _():
            issue(t + NBUF - 1, lax.rem(t + NBUF - 1, NBUF))

        pltpu.async_copy(
            rows_v.at[slot],
            out_hbm.at[pl.ds(base_b + t * GB, GB), pl.ds(0, s), pl.ds(0, 64)],
            osem)
        return carry

    lax.fori_loop(0, niter, body, 0)
    pltpu.make_async_copy(
        rows_v.at[0],
        out_hbm.at[pl.ds(0, GB), pl.ds(0, s), pl.ds(0, 64)],
        osem).wait()


def kernel(x, table):
    b, s = x.shape
    v, d = table.shape
    assert b % (NW * GB) == 0
    bpw = b // NW

    grid_kernel = pl.kernel(
        functools.partial(_emb_body, bpw, s),
        mesh=plsc.VectorSubcoreMesh(core_axis_name="c", subcore_axis_name="s"),
        out_type=jax.ShapeDtypeStruct((b, 56, 128), jnp.float32),
        scratch_types=[
            pltpu.VMEM((bpw, s), jnp.int32),
            pltpu.VMEM((NBUF, GB, s, d), jnp.float32),
            pltpu.SemaphoreType.DMA,
            pltpu.SemaphoreType.DMA,
        ],
        compiler_params=pltpu.CompilerParams(use_tc_tiling_on_sc=False),
    )

    out = grid_kernel(table, x.astype(jnp.int32))
    return out[:, :s, :d]


# speedup vs baseline: 1.1168x; 1.0532x over previous
import functools

import jax
import jax.numpy as jnp
from jax import lax
from jax.experimental import pallas as pl
from jax.experimental.pallas import tpu as pltpu
from jax.experimental.pallas import tpu_sc as plsc

NC = 2
NS = 16
NW = NC * NS
NBUF = 4
GB = 4


def _emb_body(bpw, s, table_hbm, idx_hbm, out_hbm, idx_v, rows_v, gsem, osem):
    cid = lax.axis_index("c")
    sid = lax.axis_index("s")
    wid = sid * NC + cid
    base_b = wid * bpw
    niter = bpw // GB

    pltpu.sync_copy(idx_hbm.at[pl.ds(base_b, bpw)], idx_v)

    def issue(u, slot):
        for g in range(GB):
            pltpu.async_copy(
                table_hbm.at[idx_v.at[u * GB + g]], rows_v.at[slot, g], gsem)

    for i in range(NBUF - 1):
        issue(i, i)

    def body(t, carry):
        slot = lax.rem(t, NBUF)
        for g in range(GB):
            pltpu.make_async_copy(
                table_hbm.at[idx_v.at[0]], rows_v.at[slot, g], gsem).wait()

        @pl.when(t >= 1)
        def _():
            pltpu.make_async_copy(
                rows_v.at[slot],
                out_hbm.at[pl.ds(0, GB), pl.ds(0, s), pl.ds(0, 64)],
                osem).wait()

        @pl.when(t + NBUF - 1 < niter)
        def _():
            issue(t + NBUF - 1, lax.rem(t + NBUF - 1, NBUF))

        pltpu.async_copy(
            rows_v.at[slot],
            out_hbm.at[pl.ds(base_b + t * GB, GB), pl.ds(0, s), pl.ds(0, 64)],
            osem)
        return carry

    lax.fori_loop(0, niter, body, 0)
    pltpu.make_async_copy(
        rows_v.at[0],
        out_hbm.at[pl.ds(0, GB), pl.ds(0, s), pl.ds(0, 64)],
        osem).wait()


def kernel(x, table):
    b, s = x.shape
    v, d = table.shape
    assert b % (NW * GB) == 0
    bpw = b // NW

    grid_kernel = pl.kernel(
        functools.partial(_emb_body, bpw, s),
        mesh=plsc.VectorSubcoreMesh(core_axis_name="c", subcore_axis_name="s"),
        out_type=jax.ShapeDtypeStruct((b, 56, 128), jnp.float32),
        scratch_types=[
            pltpu.VMEM((bpw, s), jnp.int32),
            pltpu.VMEM((NBUF, GB, s, d), jnp.float32),
            pltpu.SemaphoreType.DMA,
            pltpu.SemaphoreType.DMA,
        ],
        compiler_params=pltpu.CompilerParams(use_tc_tiling_on_sc=False),
    )

    t2 = jnp.pad(table, ((0, 0), (0, 2 * 64 - d))).reshape(2 * v, d)
    out = grid_kernel(t2, 2 * x.astype(jnp.int32))
    return out[:, :s, :d]
